# Initial kernel scaffold; baseline (speedup 1.0000x reference)
import jax
import jax.numpy as jnp
from jax.experimental import pallas as pl


def kernel(boxes, image_meta, p2, p3, p4, p5):
    raise NotImplementedError("write your pallas kernel here")


# speedup vs baseline: 16.5122x; 16.5122x over previous
import functools

import jax
import jax.numpy as jnp
from jax import lax
from jax.experimental import pallas as pl
from jax.experimental.pallas import tpu as pltpu
from jax.experimental.pallas import tpu_sc as plsc

POOL_H = 7
POOL_W = 7
NPX = POOL_H * POOL_W
NIDX = 2 * NPX
C = 256
NW = 32
NBOX = 1000


def _roi_body(boxes_hbm, meta_hbm, ctab_hbm, f2, f3, f4, f5, out_hbm,
              boxes_v, meta_v, ct_v, y0t, y1t, x0t, x1t, wts,
              idx_a, idx_b, rows_a, rows_b, out_v, sem_a, sem_b):
    cid = lax.axis_index("c")
    sid = lax.axis_index("s")
    wid = sid * 2 + cid
    base = wid * 31 + jnp.minimum(wid, 8)
    cnt = 31 + (wid < 8).astype(jnp.int32)

    pltpu.sync_copy(boxes_hbm, boxes_v)
    pltpu.sync_copy(meta_hbm, meta_v)
    pltpu.sync_copy(ctab_hbm, ct_v)
    mv = meta_v[pl.ds(0, 16)]
    area = mv[4] * mv[5]
    th3 = jnp.float32(224.0 * 224.0 * 0.125)
    th4 = jnp.float32(224.0 * 224.0 * 0.5)
    th5 = jnp.float32(224.0 * 224.0 * 2.0)
    lanes = lax.broadcasted_iota(jnp.int32, (16,), 0)
    lanesf = lanes.astype(jnp.float32)

    def box_body(i, carry):
        b = base + i
        bv = boxes_v[pl.ds(b * 4, 16)]
        y1 = bv[0]
        x1 = bv[1]
        y2 = bv[2]
        x2 = bv[3]
        bh = y2 - y1
        bw = x2 - x1
        hw = bh * bw * area
        lvl = (2 + (hw > th3).astype(jnp.int32)
               + (hw > th4).astype(jnp.int32)
               + (hw > th5).astype(jnp.int32))
        wdim = lax.shift_right_logical(jnp.int32(256), lvl - 2)
        wm1 = wdim - 1
        wm1f = wm1.astype(jnp.float32)

        ysv = y1 * wm1f + lanesf * (bh * wm1f * (1.0 / 6.0))
        xsv = x1 * wm1f + lanesf * (bw * wm1f * (1.0 / 6.0))
        y0i = ysv.astype(jnp.int32)
        x0i = xsv.astype(jnp.int32)
        wyv = ysv - y0i.astype(jnp.float32)
        wxv = xsv - x0i.astype(jnp.float32)
        y0c = jnp.maximum(jnp.minimum(y0i, wm1), 0)
        x0c = jnp.maximum(jnp.minimum(x0i, wm1), 0)
        y0t[...] = y0c
        y1t[...] = jnp.minimum(y0c + 1, wm1)
        x0t[...] = x0c
        x1t[...] = jnp.minimum(x0c + 1, wm1)

        for p in range(NPX):
            wy = wyv[p // POOL_W]
            wx = wxv[p % POOL_W]
            wts[p, pl.ds(0, 16)] = jnp.full((16,), (1.0 - wy) * (1.0 - wx))
            wts[p, pl.ds(16, 16)] = jnp.full((16,), (1.0 - wy) * wx)
            wts[p, pl.ds(32, 16)] = jnp.full((16,), wy * (1.0 - wx))
            wts[p, pl.ds(48, 16)] = jnp.full((16,), wy * wx)

        for ytab, idxref in ((y0t, idx_a), (y1t, idx_b)):
            for j in range(7):
                iy = ct_v[pl.ds(j * 64, 16)]
                ix = ct_v[pl.ds(j * 64 + 16, 16)]
                is_tl = ct_v[pl.ds(j * 64 + 32, 16)] > 0
                yv = plsc.load_gather(ytab, [iy])
                xv = jnp.where(is_tl,
                               plsc.load_gather(x0t, [ix]),
                               plsc.load_gather(x1t, [ix]))
                idxv = yv * wdim + xv
                if j < 6:
                    idxref[pl.ds(j * 16, 16)] = idxv
                else:
                    gcv = ct_v[pl.ds(j * 64 + 48, 16)]
                    plsc.store_scatter(idxref, [gcv], idxv, mask=lanes < 2)

        for level, fmap in ((2, f2), (3, f3), (4, f4), (5, f5)):
            @pl.when(lvl == level)
            def _():
                pltpu.async_copy(fmap.at[idx_a], rows_a, sem_a)
                pltpu.async_copy(fmap.at[idx_b], rows_b, sem_b)
        pltpu.make_async_copy(f2.at[idx_a], rows_a, sem_a).wait()
        pltpu.make_async_copy(f2.at[idx_b], rows_b, sem_b).wait()

        def px_body(p, c2):
            w00 = wts[p, pl.ds(0, 16)]
            w01 = wts[p, pl.ds(16, 16)]
            w10 = wts[p, pl.ds(32, 16)]
            w11 = wts[p, pl.ds(48, 16)]
            q = p + NPX
            for ck in range(C // 16):
                sl = pl.ds(ck * 16, 16)
                out_v[p, sl] = (rows_a[p, sl] * w00 + rows_a[q, sl] * w01
                                + rows_b[p, sl] * w10 + rows_b[q, sl] * w11)
            return c2
        lax.fori_loop(0, NPX, px_body, 0)

        pltpu.sync_copy(out_v, out_hbm.at[pl.ds(b * NPX, NPX)])
        return carry

    lax.fori_loop(0, cnt, box_body, 0)


def _make_ctab():
    import numpy as np
    rows = []
    for j in range(7):
        gc = np.minimum(np.arange(j * 16, j * 16 + 16), NIDX - 1)
        p = gc % NPX
        rows += [p // POOL_W, p % POOL_W, (gc < NPX).astype(np.int64), gc]
    return jnp.asarray(np.concatenate(rows), jnp.int32)


@jax.jit
def _roialign(boxes_flat, meta_flat, ctab, f2, f3, f4, f5):
    mesh = plsc.VectorSubcoreMesh(core_axis_name="c", subcore_axis_name="s",
                                  num_cores=2, num_subcores=16)
    return pl.kernel(
        _roi_body,
        out_type=jax.ShapeDtypeStruct((NBOX * NPX, C), jnp.float32),
        mesh=mesh,
        scratch_types=[
            pltpu.VMEM((4112,), jnp.float32),
            pltpu.VMEM((96,), jnp.float32),
            pltpu.VMEM((448,), jnp.int32),
            pltpu.VMEM((16,), jnp.int32),
            pltpu.VMEM((16,), jnp.int32),
            pltpu.VMEM((16,), jnp.int32),
            pltpu.VMEM((16,), jnp.int32),
            pltpu.VMEM((NPX, 64), jnp.float32),
            pltpu.VMEM((NIDX,), jnp.int32),
            pltpu.VMEM((NIDX,), jnp.int32),
            pltpu.VMEM((NIDX, C), jnp.float32),
            pltpu.VMEM((NIDX, C), jnp.float32),
            pltpu.VMEM((NPX, C), jnp.float32),
            pltpu.SemaphoreType.DMA,
            pltpu.SemaphoreType.DMA,
        ],
        compiler_params=pltpu.CompilerParams(use_tc_tiling_on_sc=False,
                                             needs_layout_passes=False),
    )(boxes_flat, meta_flat, ctab, f2, f3, f4, f5)


def kernel(boxes, image_meta, p2, p3, p4, p5):
    n = boxes.shape[1]
    boxes_flat = jnp.pad(boxes.reshape(-1), (0, 4112 - 4 * n))
    meta_flat = jnp.pad(image_meta.reshape(-1), (0, 96 - image_meta.size))
    out = _roialign(boxes_flat, meta_flat, _make_ctab(),
                    p2.reshape(-1, C), p3.reshape(-1, C),
                    p4.reshape(-1, C), p5.reshape(-1, C))
    return out.reshape(1, n, POOL_H, POOL_W, C)
